# baseline (speedup 1.0000x reference)
import functools

import jax
import jax.numpy as jnp
from jax import lax
from jax.experimental import pallas as pl
from jax.experimental.pallas import tpu as pltpu
from jax.experimental.pallas import tpu_sc as plsc

NC = 2
NS = 16
NW = NC * NS
D = 128
DP = D // 2
E_TOTAL = 320000
EPW = E_TOTAL // NW
CHUNK = 80
NCHUNK = EPW // CHUNK


def _dot_body(ei_hbm, ej_hbm, z_hbm, out_hbm,
              idxi_all, idxj_all, ri_a, rj_a, ri_b, rj_b, outv,
              si_a, sj_a, si_b, sj_b):
    wid = lax.axis_index("s") * NC + lax.axis_index("c")
    ebase = wid * EPW
    pltpu.sync_copy(ei_hbm.at[pl.ds(ebase, EPW)], idxi_all)
    pltpu.sync_copy(ej_hbm.at[pl.ds(ebase, EPW)], idxj_all)
    lane = lax.iota(jnp.int32, 16)

    def start(c, ri, rj, si, sj):
        pltpu.async_copy(z_hbm.at[idxi_all.at[pl.ds(c * CHUNK, CHUNK)]], ri, si)
        pltpu.async_copy(z_hbm.at[idxj_all.at[pl.ds(c * CHUNK, CHUNK)]], rj, sj)

    def wait(c, ri, rj, si, sj):
        pltpu.make_async_copy(
            z_hbm.at[idxi_all.at[pl.ds(c * CHUNK, CHUNK)]], ri, si).wait()
        pltpu.make_async_copy(
            z_hbm.at[idxj_all.at[pl.ds(c * CHUNK, CHUNK)]], rj, sj).wait()

    def pair_prod(vi32, vj32):
        pbf = plsc.bitcast(vi32, jnp.bfloat16) * plsc.bitcast(vj32, jnp.bfloat16)
        return plsc.unpack(pbf, format=plsc.PackFormat.INTERLEAVED)

    def compute(c, ri, rj):
        def group_body(g, carry):
            e_idx = lane + g * 16
            accs = [jnp.zeros((16,), jnp.float32) for _ in range(2)]
            zero = jnp.bitwise_and(e_idx, 0)
            for blk in range(DP // 16):
                for t in range(16):
                    dv = zero + (blk * 16 + t)
                    pa, pb = pair_prod(plsc.load_gather(ri, [e_idx, dv]),
                                       plsc.load_gather(rj, [e_idx, dv]))
                    accs[0] = accs[0] + pa
                    accs[1] = accs[1] + pb
            outv[pl.ds(c * CHUNK + g * 16, 16)] = accs[0] + accs[1]
            return carry

        lax.fori_loop(0, CHUNK // 16, group_body, 0)

    start(0, ri_a, rj_a, si_a, sj_a)

    def body2(t2, carry):
        t = 2 * t2
        start(t + 1, ri_b, rj_b, si_b, sj_b)
        wait(t, ri_a, rj_a, si_a, sj_a)
        compute(t, ri_a, rj_a)
        start(t + 2, ri_a, rj_a, si_a, sj_a)
        wait(t + 1, ri_b, rj_b, si_b, sj_b)
        compute(t + 1, ri_b, rj_b)
        return carry

    lax.fori_loop(0, (NCHUNK - 1) // 2, body2, 0)
    wait(NCHUNK - 1, ri_a, rj_a, si_a, sj_a)
    compute(NCHUNK - 1, ri_a, rj_a)
    pltpu.sync_copy(outv, out_hbm.at[pl.ds(ebase, EPW)])


@jax.jit
def kernel(z, edge_index):
    ei = edge_index[0].astype(jnp.int32)
    ej = edge_index[1].astype(jnp.int32)
    zp = lax.bitcast_convert_type(
        z.astype(jnp.bfloat16).reshape(z.shape[0], DP, 2), jnp.int32)
    mesh = plsc.VectorSubcoreMesh(core_axis_name="c", subcore_axis_name="s")
    f = functools.partial(
        pl.kernel,
        mesh=mesh,
        out_type=jax.ShapeDtypeStruct((E_TOTAL,), jnp.float32),
        scratch_types=[
            pltpu.VMEM((EPW,), jnp.int32),
            pltpu.VMEM((EPW,), jnp.int32),
            pltpu.VMEM((CHUNK, DP), jnp.int32),
            pltpu.VMEM((CHUNK, DP), jnp.int32),
            pltpu.VMEM((CHUNK, DP), jnp.int32),
            pltpu.VMEM((CHUNK, DP), jnp.int32),
            pltpu.VMEM((EPW,), jnp.float32),
            pltpu.SemaphoreType.DMA,
            pltpu.SemaphoreType.DMA,
            pltpu.SemaphoreType.DMA,
            pltpu.SemaphoreType.DMA,
        ],
        compiler_params=pltpu.CompilerParams(
            needs_layout_passes=False, use_tc_tiling_on_sc=False),
    )(_dot_body)
    return f(ei, ej, zp)


# speedup vs baseline: 4.5247x; 4.5247x over previous
import functools

import jax
import jax.numpy as jnp
from jax import lax
from jax.experimental import pallas as pl
from jax.experimental.pallas import tpu as pltpu
from jax.experimental.pallas import tpu_sc as plsc

NC = 2
NS = 16
NW = NC * NS
D = 128
DP = D // 2
E_TOTAL = 320000
EPW = E_TOTAL // NW
CHUNK = 400
SUB = 80
NCHUNK = EPW // CHUNK


def _dot_body(ei_hbm, ej_hbm, z_hbm, out_hbm,
              idxi_all, idxj_all, ri_a, rj_a, ri_b, rj_b, out_a, out_b,
              si_a, sj_a, si_b, sj_b, osem_a, osem_b):
    wid = lax.axis_index("s") * NC + lax.axis_index("c")
    ebase = wid * EPW
    pltpu.sync_copy(ei_hbm.at[pl.ds(ebase, EPW)], idxi_all)
    pltpu.sync_copy(ej_hbm.at[pl.ds(ebase, EPW)], idxj_all)
    lane = lax.iota(jnp.int32, 16)

    def start(c, ri, rj, si, sj):
        for k in range(CHUNK // SUB):
            pltpu.async_copy(
                z_hbm.at[idxi_all.at[pl.ds(c * CHUNK + k * SUB, SUB)]],
                ri.at[pl.ds(k * SUB, SUB)], si)
            pltpu.async_copy(
                z_hbm.at[idxj_all.at[pl.ds(c * CHUNK + k * SUB, SUB)]],
                rj.at[pl.ds(k * SUB, SUB)], sj)

    def wait(c, ri, rj, si, sj):
        for k in range(CHUNK // SUB):
            pltpu.make_async_copy(
                z_hbm.at[idxi_all.at[pl.ds(c * CHUNK + k * SUB, SUB)]],
                ri.at[pl.ds(k * SUB, SUB)], si).wait()
            pltpu.make_async_copy(
                z_hbm.at[idxj_all.at[pl.ds(c * CHUNK + k * SUB, SUB)]],
                rj.at[pl.ds(k * SUB, SUB)], sj).wait()

    def out_start(c, outb, osem):
        pltpu.async_copy(outb, out_hbm.at[pl.ds(ebase + c * CHUNK, CHUNK)],
                         osem)

    def out_wait(c, outb, osem):
        pltpu.make_async_copy(
            outb, out_hbm.at[pl.ds(ebase + c * CHUNK, CHUNK)], osem).wait()

    def pair_prod(vi32, vj32):
        pbf = plsc.bitcast(vi32, jnp.bfloat16) * plsc.bitcast(vj32, jnp.bfloat16)
        return plsc.unpack(pbf, format=plsc.PackFormat.INTERLEAVED)

    def compute(c, ri, rj, outb):
        def group_body(g, carry):
            e_idx = lane + g * 16
            accs = [jnp.zeros((16,), jnp.float32) for _ in range(2)]
            for blk in range(DP // 16):
                for t in range(16):
                    dv = (lane ^ t) + blk * 16
                    pa, pb = pair_prod(plsc.load_gather(ri, [e_idx, dv]),
                                       plsc.load_gather(rj, [e_idx, dv]))
                    accs[0] = accs[0] + pa
                    accs[1] = accs[1] + pb
            outb[pl.ds(g * 16, 16)] = accs[0] + accs[1]
            return carry

        lax.fori_loop(0, CHUNK // 16, group_body, 0)

    start(0, ri_a, rj_a, si_a, sj_a)

    def body2(t2, carry):
        t = 2 * t2

        @pl.when(t2 > 0)
        def _():
            out_wait(t - 2, out_a, osem_a)

        start(t + 1, ri_b, rj_b, si_b, sj_b)
        wait(t, ri_a, rj_a, si_a, sj_a)
        compute(t, ri_a, rj_a, out_a)
        out_start(t, out_a, osem_a)

        @pl.when(t2 > 0)
        def _():
            out_wait(t - 1, out_b, osem_b)

        start(t + 2, ri_a, rj_a, si_a, sj_a)
        wait(t + 1, ri_b, rj_b, si_b, sj_b)
        compute(t + 1, ri_b, rj_b, out_b)
        out_start(t + 1, out_b, osem_b)
        return carry

    lax.fori_loop(0, (NCHUNK - 1) // 2, body2, 0)
    out_wait(NCHUNK - 3, out_a, osem_a)
    wait(NCHUNK - 1, ri_a, rj_a, si_a, sj_a)
    compute(NCHUNK - 1, ri_a, rj_a, out_a)
    out_start(NCHUNK - 1, out_a, osem_a)
    out_wait(NCHUNK - 2, out_b, osem_b)
    out_wait(NCHUNK - 1, out_a, osem_a)


@jax.jit
def kernel(z, edge_index):
    ei = edge_index[0].astype(jnp.int32)
    ej = edge_index[1].astype(jnp.int32)
    zp = lax.bitcast_convert_type(
        z.astype(jnp.bfloat16).reshape(z.shape[0], DP, 2), jnp.int32)
    mesh = plsc.VectorSubcoreMesh(core_axis_name="c", subcore_axis_name="s")
    f = functools.partial(
        pl.kernel,
        mesh=mesh,
        out_type=jax.ShapeDtypeStruct((E_TOTAL,), jnp.float32),
        scratch_types=[
            pltpu.VMEM((EPW,), jnp.int32),
            pltpu.VMEM((EPW,), jnp.int32),
            pltpu.VMEM((CHUNK, DP), jnp.int32),
            pltpu.VMEM((CHUNK, DP), jnp.int32),
            pltpu.VMEM((CHUNK, DP), jnp.int32),
            pltpu.VMEM((CHUNK, DP), jnp.int32),
            pltpu.VMEM((CHUNK,), jnp.float32),
            pltpu.VMEM((CHUNK,), jnp.float32),
            pltpu.SemaphoreType.DMA,
            pltpu.SemaphoreType.DMA,
            pltpu.SemaphoreType.DMA,
            pltpu.SemaphoreType.DMA,
            pltpu.SemaphoreType.DMA,
            pltpu.SemaphoreType.DMA,
        ],
        compiler_params=pltpu.CompilerParams(
            needs_layout_passes=False, use_tc_tiling_on_sc=False),
    )(_dot_body)
    return f(ei, ej, zp)
